# Initial kernel scaffold; baseline (speedup 1.0000x reference)
import jax
import jax.numpy as jnp
from jax.experimental import pallas as pl


def kernel(input, table):
    raise NotImplementedError("write your pallas kernel here")


# speedup vs baseline: 1.5611x; 1.5611x over previous
import functools

import jax
import jax.numpy as jnp
from jax import lax
from jax.experimental import pallas as pl
from jax.experimental.pallas import tpu as pltpu
from jax.experimental.pallas import tpu_sc as plsc

_INFO = plsc.get_sparse_core_info()
_NC = _INFO.num_cores
_NS = _INFO.num_subcores
_NW = _NC * _NS


def _emb_lookup(idx_flat, table, chunk, n_chunks):
    B = idx_flat.shape[0]
    D = table.shape[1]
    b_per_w = B // _NW
    mesh = plsc.VectorSubcoreMesh(core_axis_name="c", subcore_axis_name="s")

    @functools.partial(
        pl.kernel,
        mesh=mesh,
        out_type=jax.ShapeDtypeStruct((B, D), jnp.float32),
        scratch_types=[
            pltpu.VMEM((chunk,), jnp.int32),
            pltpu.VMEM((chunk, D), jnp.float32),
            pltpu.SemaphoreType.DMA,
        ],
        compiler_params=pltpu.CompilerParams(use_tc_tiling_on_sc=False),
    )
    def emb(idx_hbm, tab_hbm, out_hbm, idx_v, rows_v, sem):
        wid = lax.axis_index("s") * _NC + lax.axis_index("c")
        base = wid * b_per_w
        for i in range(n_chunks):
            off = base + i * chunk
            pltpu.sync_copy(idx_hbm.at[pl.ds(off, chunk)], idx_v)
            pltpu.async_copy(tab_hbm.at[idx_v], rows_v, sem).wait()
            pltpu.sync_copy(rows_v, out_hbm.at[pl.ds(off, chunk)])

    return emb(idx_flat, table)


def kernel(input, table):
    n, k = input.shape
    D = table.shape[1]
    idx_flat = input.reshape(-1).astype(jnp.int32)
    B = idx_flat.shape[0]
    chunk = 1664
    n_chunks = (B // _NW) // chunk
    out = _emb_lookup(idx_flat, table, chunk, n_chunks)
    return out.reshape(n, k, D)


# speedup vs baseline: 1.5763x; 1.0097x over previous
import functools

import jax
import jax.numpy as jnp
from jax import lax
from jax.experimental import pallas as pl
from jax.experimental.pallas import tpu as pltpu
from jax.experimental.pallas import tpu_sc as plsc

_INFO = plsc.get_sparse_core_info()
_NC = _INFO.num_cores
_NS = _INFO.num_subcores
_NW = _NC * _NS


def _emb_lookup(idx_flat, table, chunk, n_chunks):
    B = idx_flat.shape[0]
    D = table.shape[1]
    b_per_w = B // _NW
    nbuf = 2
    mesh = plsc.VectorSubcoreMesh(core_axis_name="c", subcore_axis_name="s")

    @functools.partial(
        pl.kernel,
        mesh=mesh,
        out_type=jax.ShapeDtypeStruct((B, D), jnp.float32),
        scratch_types=[
            pltpu.VMEM((b_per_w,), jnp.int32),
            pltpu.VMEM((nbuf, chunk, D), jnp.float32),
            pltpu.SemaphoreType.DMA((nbuf,)),
            pltpu.SemaphoreType.DMA((nbuf,)),
        ],
        compiler_params=pltpu.CompilerParams(use_tc_tiling_on_sc=False),
    )
    def emb(idx_hbm, tab_hbm, out_hbm, idx_v, rows_v, gsem, ssem):
        wid = lax.axis_index("s") * _NC + lax.axis_index("c")
        base = wid * b_per_w
        pltpu.sync_copy(idx_hbm.at[pl.ds(base, b_per_w)], idx_v)

        def gather_start(i):
            b = i % nbuf
            pltpu.make_async_copy(
                tab_hbm.at[idx_v.at[pl.ds(i * chunk, chunk)]],
                rows_v.at[b],
                gsem.at[b],
            ).start()

        def gather_wait(i):
            b = i % nbuf
            pltpu.make_async_copy(
                tab_hbm.at[idx_v.at[pl.ds(i * chunk, chunk)]],
                rows_v.at[b],
                gsem.at[b],
            ).wait()

        def store_start(i):
            b = i % nbuf
            pltpu.make_async_copy(
                rows_v.at[b],
                out_hbm.at[pl.ds(base + i * chunk, chunk)],
                ssem.at[b],
            ).start()

        def store_wait(i):
            b = i % nbuf
            pltpu.make_async_copy(
                rows_v.at[b],
                out_hbm.at[pl.ds(base + i * chunk, chunk)],
                ssem.at[b],
            ).wait()

        for i in range(min(nbuf, n_chunks)):
            gather_start(i)
        for i in range(n_chunks):
            gather_wait(i)
            store_start(i)
            if i + nbuf < n_chunks:
                store_wait(i)
                gather_start(i + nbuf)
        for i in range(max(0, n_chunks - nbuf), n_chunks):
            store_wait(i)

    return emb(idx_flat, table)


def kernel(input, table):
    n, k = input.shape
    D = table.shape[1]
    idx_flat = input.reshape(-1).astype(jnp.int32)
    B = idx_flat.shape[0]
    chunk = 1664
    n_chunks = (B // _NW) // chunk
    out = _emb_lookup(idx_flat, table, chunk, n_chunks)
    return out.reshape(n, k, D)


# speedup vs baseline: 1.5815x; 1.0033x over previous
import functools

import jax
import jax.numpy as jnp
from jax import lax
from jax.experimental import pallas as pl
from jax.experimental.pallas import tpu as pltpu
from jax.experimental.pallas import tpu_sc as plsc

_INFO = plsc.get_sparse_core_info()
_NC = _INFO.num_cores
_NS = _INFO.num_subcores
_NW = _NC * _NS


def _emb_lookup(idx_flat, table, chunk, n_chunks, nbuf):
    B = idx_flat.shape[0]
    D = table.shape[1]
    b_per_w = B // _NW
    mesh = plsc.VectorSubcoreMesh(core_axis_name="c", subcore_axis_name="s")

    @functools.partial(
        pl.kernel,
        mesh=mesh,
        out_type=jax.ShapeDtypeStruct((B, D), jnp.float32),
        scratch_types=[
            pltpu.VMEM((b_per_w,), jnp.int32),
            pltpu.VMEM((nbuf, chunk, D), jnp.float32),
            pltpu.SemaphoreType.DMA((nbuf,)),
            pltpu.SemaphoreType.DMA((nbuf,)),
        ],
        compiler_params=pltpu.CompilerParams(use_tc_tiling_on_sc=False),
    )
    def emb(idx_hbm, tab_hbm, out_hbm, idx_v, rows_v, gsem, ssem):
        wid = lax.axis_index("s") * _NC + lax.axis_index("c")
        base = wid * b_per_w
        pltpu.sync_copy(idx_hbm.at[pl.ds(base, b_per_w)], idx_v)

        def gather_start(i):
            b = i % nbuf
            pltpu.make_async_copy(
                tab_hbm.at[idx_v.at[pl.ds(i * chunk, chunk)]],
                rows_v.at[b],
                gsem.at[b],
            ).start()

        def gather_wait(i):
            b = i % nbuf
            pltpu.make_async_copy(
                tab_hbm.at[idx_v.at[pl.ds(i * chunk, chunk)]],
                rows_v.at[b],
                gsem.at[b],
            ).wait()

        def store_start(i):
            b = i % nbuf
            pltpu.make_async_copy(
                rows_v.at[b],
                out_hbm.at[pl.ds(base + i * chunk, chunk)],
                ssem.at[b],
            ).start()

        def store_wait(i):
            b = i % nbuf
            pltpu.make_async_copy(
                rows_v.at[b],
                out_hbm.at[pl.ds(base + i * chunk, chunk)],
                ssem.at[b],
            ).wait()

        for i in range(min(nbuf, n_chunks)):
            gather_start(i)
        for i in range(n_chunks):
            gather_wait(i)
            store_start(i)
            if i + nbuf < n_chunks:
                store_wait(i)
                gather_start(i + nbuf)
        for i in range(max(0, n_chunks - nbuf), n_chunks):
            store_wait(i)

    return emb(idx_flat, table)


def kernel(input, table):
    n, k = input.shape
    D = table.shape[1]
    idx_flat = input.reshape(-1).astype(jnp.int32)
    B = idx_flat.shape[0]
    chunk = 832
    nbuf = 4
    n_chunks = (B // _NW) // chunk
    out = _emb_lookup(idx_flat, table, chunk, n_chunks, nbuf)
    return out.reshape(n, k, D)
